# baseline (speedup 1.0000x reference)
import functools

import jax
import jax.numpy as jnp
from jax import lax
from jax.experimental import pallas as pl
from jax.experimental.pallas import tpu as pltpu
from jax.experimental.pallas import tpu_sc as plsc

_NTOKEN = 1000
_NC = 2
_NS = 16
_NW = _NC * _NS
_LANES = 16

_CHUNK_ROWS = 32
_CHUNK_WORDS = _CHUNK_ROWS * _NTOKEN


def _poke(buf, idx_v, chunk, iota, val):
    off = chunk * _CHUNK_ROWS
    for j in range(_CHUNK_ROWS // _LANES):
        xv = idx_v[pl.ds(off + j * _LANES, _LANES)]
        pos = (j * _LANES + iota) * _NTOKEN + xv
        plsc.store_scatter(buf, [pos], val)


def _onehot_sc(x_hbm, out_hbm, idx_v, buf0, buf1, sem0, sem1, *, rows_per_w):
    wid = lax.axis_index("s") * _NC + lax.axis_index("c")
    nchunks = rows_per_w // _CHUNK_ROWS
    row_base = wid * rows_per_w
    word_base = row_base * _NTOKEN

    pltpu.sync_copy(x_hbm.at[pl.ds(row_base, rows_per_w)], idx_v)

    iota = lax.iota(jnp.int32, _LANES)
    ones = jnp.full((_LANES,), 1.0, jnp.float32)
    zeros = jnp.zeros((_LANES,), jnp.float32)

    def _memset(i, c):
        buf0[pl.ds(i * _LANES, _LANES)] = zeros
        buf1[pl.ds(i * _LANES, _LANES)] = zeros
        return c

    lax.fori_loop(0, _CHUNK_WORDS // _LANES, _memset, 0)

    bufs = (buf0, buf1)
    sems = (sem0, sem1)

    def _dst(chunk):
        return out_hbm.at[pl.ds(word_base + chunk * _CHUNK_WORDS, _CHUNK_WORDS)]

    for b in range(2):
        _poke(bufs[b], idx_v, b, iota, ones)
        pltpu.async_copy(bufs[b], _dst(b), sems[b])

    def _step(g, c):
        for b in range(2):
            chunk = 2 * g + b
            prev = chunk - 2
            pltpu.make_async_copy(bufs[b], _dst(prev), sems[b]).wait()
            _poke(bufs[b], idx_v, prev, iota, zeros)
            _poke(bufs[b], idx_v, chunk, iota, ones)
            pltpu.async_copy(bufs[b], _dst(chunk), sems[b])
        return c

    lax.fori_loop(1, nchunks // 2, _step, 0)

    for b in range(2):
        pltpu.make_async_copy(bufs[b], _dst(nchunks - 2 + b), sems[b]).wait()


def kernel(x):
    B, L = x.shape
    rows = B * L
    rows_per_w = rows // _NW
    xf = x.reshape(rows)

    body = functools.partial(_onehot_sc, rows_per_w=rows_per_w)
    body.__name__ = "_onehot_sc"

    out = pl.kernel(
        body,
        mesh=plsc.VectorSubcoreMesh(core_axis_name="c", subcore_axis_name="s"),
        compiler_params=pltpu.CompilerParams(needs_layout_passes=False),
        out_type=jax.ShapeDtypeStruct((rows * _NTOKEN,), jnp.float32),
        scratch_types=[
            pltpu.VMEM((rows_per_w,), jnp.int32),
            pltpu.VMEM((_CHUNK_WORDS,), jnp.float32),
            pltpu.VMEM((_CHUNK_WORDS,), jnp.float32),
            pltpu.SemaphoreType.DMA,
            pltpu.SemaphoreType.DMA,
        ],
    )(xf)
    return out.reshape(B, L, _NTOKEN)


# speedup vs baseline: 1.8886x; 1.8886x over previous
import functools

import jax
import jax.numpy as jnp
from jax import lax
from jax.experimental import pallas as pl
from jax.experimental.pallas import tpu as pltpu
from jax.experimental.pallas import tpu_sc as plsc

_NTOKEN = 1000
_NC = 2
_NS = 16
_NW = _NC * _NS
_LANES = 16


def _poke(buf, idx_v, b_local, L, iota, val):
    for j in range(pl.cdiv(L, _LANES)):
        l = j * _LANES + iota
        m = l < L
        xv = plsc.load_gather(idx_v, [b_local * L + jnp.where(m, l, 0)])
        plsc.store_scatter(buf, [l, xv], val, mask=m)


def _onehot_sc(x_hbm, out_hbm, idx_v, buf0, buf1, sem0, sem1, *, L, b_per_w):
    wid = lax.axis_index("s") * _NC + lax.axis_index("c")
    b_base = wid * b_per_w

    pltpu.sync_copy(x_hbm.at[pl.ds(b_base * L, b_per_w * L)], idx_v)

    iota = lax.iota(jnp.int32, _LANES)
    ones = jnp.full((_LANES,), 1.0, jnp.float32)
    zeros = jnp.zeros((_LANES,), jnp.float32)

    offs = [k * _LANES for k in range(_NTOKEN // _LANES)] + [_NTOKEN - _LANES]

    def _memset(l, c):
        for o in offs:
            buf0[l, pl.ds(o, _LANES)] = zeros
            buf1[l, pl.ds(o, _LANES)] = zeros
        return c

    lax.fori_loop(0, L, _memset, 0)

    bufs = (buf0, buf1)
    sems = (sem0, sem1)

    for r in range(2):
        _poke(bufs[r], idx_v, r, L, iota, ones)
        pltpu.async_copy(bufs[r], out_hbm.at[b_base + r], sems[r])

    def _step(g, c):
        for r in range(2):
            b_local = 2 * g + r
            prev = b_local - 2
            pltpu.make_async_copy(bufs[r], out_hbm.at[b_base + prev], sems[r]).wait()
            _poke(bufs[r], idx_v, prev, L, iota, zeros)
            _poke(bufs[r], idx_v, b_local, L, iota, ones)
            pltpu.async_copy(bufs[r], out_hbm.at[b_base + b_local], sems[r])
        return c

    lax.fori_loop(1, b_per_w // 2, _step, 0)

    for r in range(2):
        pltpu.make_async_copy(
            bufs[r], out_hbm.at[b_base + b_per_w - 2 + r], sems[r]
        ).wait()


def kernel(x):
    B, L = x.shape
    b_per_w = B // _NW
    xf = x.reshape(B * L)

    body = functools.partial(_onehot_sc, L=L, b_per_w=b_per_w)
    body.__name__ = "_onehot_sc"

    return pl.kernel(
        body,
        mesh=plsc.VectorSubcoreMesh(core_axis_name="c", subcore_axis_name="s"),
        compiler_params=pltpu.CompilerParams(needs_layout_passes=False),
        out_type=jax.ShapeDtypeStruct((B, L, _NTOKEN), jnp.float32),
        scratch_types=[
            pltpu.VMEM((b_per_w * L,), jnp.int32),
            pltpu.VMEM((L, _NTOKEN), jnp.float32),
            pltpu.VMEM((L, _NTOKEN), jnp.float32),
            pltpu.SemaphoreType.DMA,
            pltpu.SemaphoreType.DMA,
        ],
    )(xf)


# speedup vs baseline: 1.9004x; 1.0063x over previous
import functools

import jax
import jax.numpy as jnp
from jax import lax
from jax.experimental import pallas as pl
from jax.experimental.pallas import tpu as pltpu
from jax.experimental.pallas import tpu_sc as plsc

_NTOKEN = 1000
_NC = 2
_NS = 16
_NW = _NC * _NS
_LANES = 16


def _poke(buf, idx_v, b_local, L, iota, val):
    for j in range(pl.cdiv(L, _LANES)):
        l = j * _LANES + iota
        m = l < L
        xv = plsc.load_gather(idx_v, [b_local * L + jnp.where(m, l, 0)])
        plsc.store_scatter(buf, [l, xv], val, mask=m)


def _onehot_sc(x_hbm, out_hbm, idx_v, buf0, buf1, sem0, sem1, *, L, b_per_w):
    wid = lax.axis_index("s") * _NC + lax.axis_index("c")
    b_base = wid * b_per_w

    pltpu.sync_copy(x_hbm.at[pl.ds(b_base * L, b_per_w * L)], idx_v)

    iota = lax.iota(jnp.int32, _LANES)
    ones = jnp.full((_LANES,), 1.0, jnp.float32)
    zeros = jnp.zeros((_LANES,), jnp.float32)

    offs = [k * _LANES for k in range(_NTOKEN // _LANES)] + [_NTOKEN - _LANES]

    def _memset(l, c):
        for o in offs:
            buf0[l, pl.ds(o, _LANES)] = zeros
            buf1[l, pl.ds(o, _LANES)] = zeros
        return c

    lax.fori_loop(0, L, _memset, 0)

    bufs = (buf0, buf1)
    sems = (sem0, sem1)

    for r in range(2):
        _poke(bufs[r], idx_v, r, L, iota, ones)
        pltpu.async_copy(bufs[r], out_hbm.at[b_base + r], sems[r])

    def _step(g, c):
        for r in range(2):
            b_local = 2 * g + r
            prev = b_local - 2
            pltpu.make_async_copy(bufs[r], out_hbm.at[b_base + prev], sems[r]).wait()
            _poke(bufs[r], idx_v, prev, L, iota, zeros)
            _poke(bufs[r], idx_v, b_local, L, iota, ones)
            pltpu.async_copy(bufs[r], out_hbm.at[b_base + b_local], sems[r])
        return c

    lax.fori_loop(1, b_per_w // 2, _step, 0)

    for r in range(2):
        pltpu.make_async_copy(
            bufs[r], out_hbm.at[b_base + b_per_w - 2 + r], sems[r]
        ).wait()


def kernel(x):
    B, L = x.shape
    b_per_w = B // _NW
    xf = x.reshape(B * L)

    body = functools.partial(_onehot_sc, L=L, b_per_w=b_per_w)
    body.__name__ = "_onehot_sc"

    return pl.kernel(
        body,
        mesh=plsc.VectorSubcoreMesh(core_axis_name="c", subcore_axis_name="s"),
        compiler_params=pltpu.CompilerParams(
            needs_layout_passes=False, skip_device_barrier=True
        ),
        out_type=jax.ShapeDtypeStruct((B, L, _NTOKEN), jnp.float32),
        scratch_types=[
            pltpu.VMEM((b_per_w * L,), jnp.int32),
            pltpu.VMEM((L, _NTOKEN), jnp.float32),
            pltpu.VMEM((L, _NTOKEN), jnp.float32),
            pltpu.SemaphoreType.DMA,
            pltpu.SemaphoreType.DMA,
        ],
    )(xf)
